# baseline (speedup 1.0000x reference)
import functools

import jax
import jax.numpy as jnp
from jax import lax
from jax.experimental import pallas as pl
from jax.experimental.pallas import tpu as pltpu
from jax.experimental.pallas import tpu_sc as plsc

_B = 16384
_A_MAIN = 99968
_T_MAIN = 999936
_A_SHIFT = 7
_T_SHIFT = 9
_A_NCH = _A_MAIN >> _A_SHIFT
_T_NCH = _T_MAIN >> _T_SHIFT


def _sc_gather(author, comment, a3, t3):
    info = plsc.get_sparse_core_info()
    nc, ns = info.num_cores, info.num_subcores
    mesh = plsc.VectorSubcoreMesh(core_axis_name="c", subcore_axis_name="s")

    @functools.partial(
        pl.kernel,
        mesh=mesh,
        compiler_params=pltpu.CompilerParams(needs_layout_passes=False),
        out_type=[
            jax.ShapeDtypeStruct((_B, 128), jnp.float32),
            jax.ShapeDtypeStruct((_B, 128), jnp.float32),
        ],
        scratch_types=[
            pltpu.VMEM((_B,), jnp.int32),
            pltpu.VMEM((_B + 16,), jnp.int32),
            pltpu.VMEM((_B + 16,), jnp.int32),
            pltpu.VMEM((8, 8, 128), jnp.float32),
            pltpu.VMEM((16, 8, 512), jnp.float32),
            pltpu.VMEM((16, 128), jnp.float32),
            pltpu.VMEM((16,), jnp.int32),
            pltpu.SemaphoreType.DMA,
            pltpu.SemaphoreType.DMA,
            pltpu.SemaphoreType.DMA,
        ],
    )
    def gather_kernel(a_hbm, c_hbm, a3h, t3h, out_a, out_t,
                      idxv, lsti, lstp, buf_a, buf_t, ring, tmpi,
                      sem_sa, sem_sb, sem_out):
        wid = lax.axis_index("s") * nc + lax.axis_index("c")
        iota16 = lax.broadcasted_iota(jnp.int32, (16,), 0)

        def phase(idx_hbm, tbl, buf, out, shift, nch, vmax, ngroups, nadim,
                  k_out0):
            chunk = 1 << shift
            pltpu.sync_copy(idx_hbm, idxv)

            def p1(j, n):
                v = idxv[pl.ds(16 * j, 16)]
                pos = iota16 + 16 * j
                m = (((v >> shift) & 31) == wid) & (v < vmax)
                mi = jnp.where(m, 1, 0)
                km = jnp.sum(mi)

                @pl.when(km > 0)
                def _scatter():
                    tgt = n + plsc.cumsum(mi) - mi
                    plsc.store_scatter(lsti.at[:], [tgt], v, mask=m)
                    plsc.store_scatter(lstp.at[:], [tgt], pos, mask=m)

                return n + km

            n_my = lax.fori_loop(0, _B // 16, p1, 0)
            lsti[pl.ds(n_my, 16)] = jnp.full((16,), -1, jnp.int32)
            nv = (n_my + 15) >> 4
            nch_w = (nch - wid + 31) // 32
            fbase = []
            for g in range(ngroups):
                f = iota16 + 16 * g
                fbase.append((f >> 3, f & 7))

            def issue(ci, half, sem):
                base = pl.multiple_of(ci << shift, 128)
                for a in range(nadim):
                    pltpu.async_copy(
                        tbl.at[a, :, pl.ds(base, chunk)],
                        buf.at[half * nadim + a], sem)

            def wait_half(half, sem):
                for a in range(nadim):
                    pltpu.make_async_copy(
                        tbl.at[0, :, pl.ds(0, chunk)],
                        buf.at[half * nadim + a], sem).wait()

            def extract(c, half, k_in):
                def scan_body(j, k2):
                    v = lsti[pl.ds(16 * j, 16)]
                    vp = lstp[pl.ds(16 * j, 16)]
                    m = (v >> shift) == c
                    mi = jnp.where(m, 1, 0)
                    km = jnp.sum(mi)

                    @pl.when(km > 0)
                    def _scatter():
                        tgt = plsc.cumsum(mi) - mi
                        pk = (v & (chunk - 1)) | (vp << shift)
                        plsc.store_scatter(tmpi.at[:], [tgt], pk, mask=m)

                    def ser(r, k3):
                        vt = tmpi[...]
                        sel = jnp.where(iota16 == r, 1, 0)
                        pk = jnp.sum(vt * sel)
                        off = pk & (chunk - 1)
                        pos = pk >> shift
                        slot = k3 & 15

                        @pl.when(k3 >= 16)
                        def _drain():
                            pltpu.make_async_copy(
                                out.at[pl.ds(0, 1)],
                                ring.at[pl.ds(slot, 1)], sem_out).wait()

                        offv = jnp.full((16,), off, jnp.int32)
                        for g in range(ngroups):
                            av, bv = fbase[g]
                            vals = plsc.load_gather(
                                buf.at[:, :, :],
                                [av + half * nadim, bv, offv])
                            ring[slot, pl.ds(16 * g, 16)] = vals
                        pltpu.async_copy(
                            ring.at[pl.ds(slot, 1)],
                            out.at[pl.ds(pos, 1)], sem_out)
                        return k3 + 1

                    return lax.fori_loop(0, km, ser, k2)

                return lax.fori_loop(0, nv, scan_body, k_in)

            issue(wid, 0, sem_sa)
            npairs = (nch_w + 1) // 2

            def pair_body(p2, k_out):
                c0 = wid + 64 * p2
                c1 = c0 + 32
                c2 = c0 + 64
                issue(jnp.minimum(c1, nch - 1), 1, sem_sb)
                wait_half(0, sem_sa)
                k_out = extract(c0, 0, k_out)
                issue(jnp.minimum(c2, nch - 1), 0, sem_sa)
                wait_half(1, sem_sb)
                return extract(c1, 1, k_out)

            k_out = lax.fori_loop(0, npairs, pair_body, k_out0)
            wait_half(0, sem_sa)
            return k_out

        k1 = phase(a_hbm, a3h, buf_a, out_a, _A_SHIFT, _A_NCH, _A_MAIN,
                   2, 4, 0)
        k2 = phase(c_hbm, t3h, buf_t, out_t, _T_SHIFT, _T_NCH, _T_MAIN,
                   4, 8, k1)
        for i in range(8):
            @pl.when(i < jnp.minimum(k2, 8))
            def _final_drain():
                pltpu.make_async_copy(
                    out_t.at[pl.ds(0, 1)], ring.at[pl.ds(0, 1)],
                    sem_out).wait()

    return gather_kernel(author, comment, a3, t3)


def _tc_s_body(su, s_emb, s_w, s_b, out):
    f32 = jnp.float32
    blk = su.shape[0]
    svocab = s_emb.shape[0]
    ia_s = lax.broadcasted_iota(jnp.int32, (blk, svocab), 1)
    oh_s = ((su[...][:, None] + ia_s * 0) == ia_s).astype(f32)
    se = jnp.dot(oh_s, s_emb[...], preferred_element_type=f32)
    out[...] = jnp.dot(se, s_w[...], preferred_element_type=f32) + s_b[...]


def _tc_s(su, s_emb, s_w, s_b):
    blk = 2048
    grid = _B // blk

    def full(x):
        return pl.BlockSpec(x.shape, lambda i: (0,) * x.ndim)

    return pl.pallas_call(
        _tc_s_body,
        grid=(grid,),
        in_specs=[pl.BlockSpec((blk,), lambda i: (i,)),
                  full(s_emb), full(s_w), full(s_b)],
        out_specs=pl.BlockSpec((blk, 50), lambda i: (i, 0)),
        out_shape=jax.ShapeDtypeStruct((_B, 50), jnp.float32),
    )(su, s_emb, s_w, s_b)


def _tc_body(au, cu, ga, gt, sru, a_tail, t_tail, a_w, a_b,
             t_w1, t_w2, t_b1, t_b2, l1a, l1c, l1_b, l2_w, l2_b, out):
    f32 = jnp.float32
    blk = au.shape[0]
    au_, cu_ = au[...], cu[...]
    ga_ = ga[...][:, :32]
    gt_ = gt[...][:, :64]
    ia32 = lax.broadcasted_iota(jnp.int32, (blk, 32), 1)
    au2 = au_[:, None] + ia32 * 0
    oh_a = ((au2 - _A_MAIN) == ia32).astype(f32)
    ae = jnp.where(au2 >= _A_MAIN,
                   jnp.dot(oh_a, a_tail[...], preferred_element_type=f32),
                   ga_)
    ia64 = lax.broadcasted_iota(jnp.int32, (blk, 64), 1)
    cu2 = cu_[:, None] + ia64 * 0
    oh_t = ((cu2 - _T_MAIN) == ia64).astype(f32)
    te = jnp.where(cu2 >= _T_MAIN,
                   jnp.dot(oh_t, t_tail[...], preferred_element_type=f32),
                   gt_)
    ar = jnp.dot(ae, a_w[...], preferred_element_type=f32) + a_b[...]
    sr = sru[...]
    cr1 = jnp.dot(te, t_w1[...], preferred_element_type=f32) + t_b1[...]
    cr2 = jnp.dot(te, t_w2[...], preferred_element_type=f32) + t_b2[...]
    m = (jnp.dot(ar * cr1, l1a[...], preferred_element_type=f32)
         + jnp.dot(sr * cr2, l1c[...], preferred_element_type=f32)
         + l1_b[...])
    o = jnp.dot(m, l2_w[...], preferred_element_type=f32) + l2_b[...]
    out[...] = o[:, 0]


def _tc_dense(au, cu, ga, gt, sru, *weights):
    blk = 2048
    grid = _B // blk

    def full(x):
        return pl.BlockSpec(x.shape, lambda i: (0,) * x.ndim)

    vec = pl.BlockSpec((blk,), lambda i: (i,))
    mat = pl.BlockSpec((blk, 128), lambda i: (i, 0))
    return pl.pallas_call(
        _tc_body,
        grid=(grid,),
        in_specs=[vec, vec, mat, mat,
                  pl.BlockSpec((blk, 50), lambda i: (i, 0)),
                  *[full(w) for w in weights]],
        out_specs=pl.BlockSpec((blk,), lambda i: (i,)),
        out_shape=jax.ShapeDtypeStruct((_B,), jnp.float32),
    )(au, cu, ga, gt, sru, *weights)


def kernel(author, subreddit, comment, A_emb, A_W, A_b, S_emb, S_W, S_b,
           T_emb, T_W, T_b, L1_W, L1_b, L2_W, L2_b):
    author = author.astype(jnp.int32)
    subreddit = subreddit.astype(jnp.int32)
    comment = comment.astype(jnp.int32)
    a3 = A_emb.T.reshape(4, 8, A_emb.shape[0])
    t3 = T_emb.T.reshape(8, 8, T_emb.shape[0])
    a_tail = A_emb[_A_MAIN:]
    t_tail = T_emb[_T_MAIN:]
    sru = _tc_s(subreddit, S_emb, S_W, S_b)
    ga, gt = _sc_gather(author, comment, a3, t3)
    t_w1, t_w2 = T_W[:, :50], T_W[:, 50:]
    t_b1, t_b2 = T_b[:50], T_b[50:]
    l1a, l1c = L1_W[:50, :], L1_W[50:, :]
    return _tc_dense(author, comment, ga, gt, sru,
                     a_tail, t_tail, A_W, A_b,
                     t_w1, t_w2, t_b1, t_b2, l1a, l1c, L1_b, L2_W, L2_b)


# speedup vs baseline: 1.1201x; 1.1201x over previous
import functools

import jax
import jax.numpy as jnp
from jax import lax
from jax.experimental import pallas as pl
from jax.experimental.pallas import tpu as pltpu
from jax.experimental.pallas import tpu_sc as plsc

_B = 16384
_A_MAIN = 99968
_T_MAIN = 999936
_A_SHIFT = 7
_T_SHIFT = 9
_A_NCH = _A_MAIN >> _A_SHIFT
_T_NCH = _T_MAIN >> _T_SHIFT


def _sc_gather(author, comment, a3, t3):
    info = plsc.get_sparse_core_info()
    nc, ns = info.num_cores, info.num_subcores
    mesh = plsc.VectorSubcoreMesh(core_axis_name="c", subcore_axis_name="s")

    @functools.partial(
        pl.kernel,
        mesh=mesh,
        compiler_params=pltpu.CompilerParams(needs_layout_passes=False),
        out_type=[
            jax.ShapeDtypeStruct((_B, 128), jnp.float32),
            jax.ShapeDtypeStruct((_B, 128), jnp.float32),
        ],
        scratch_types=[
            pltpu.VMEM((_B,), jnp.int32),
            pltpu.VMEM((_B + 16,), jnp.int32),
            pltpu.VMEM((_B + 16,), jnp.int32),
            pltpu.VMEM((8, 8, 128), jnp.float32),
            pltpu.VMEM((16, 8, 512), jnp.float32),
            pltpu.VMEM((16, 128), jnp.float32),
            pltpu.VMEM((16,), jnp.int32),
            pltpu.SemaphoreType.DMA,
            pltpu.SemaphoreType.DMA,
            pltpu.SemaphoreType.DMA,
        ],
    )
    def gather_kernel(a_hbm, c_hbm, a3h, t3h, out_a, out_t,
                      idxv, lsti, lstp, buf_a, buf_t, ring, tmpi,
                      sem_sa, sem_sb, sem_out):
        wid = lax.axis_index("s") * nc + lax.axis_index("c")
        iota16 = lax.broadcasted_iota(jnp.int32, (16,), 0)

        def phase(idx_hbm, tbl, buf, out, shift, nch, vmax, ngroups, nadim,
                  k_out0):
            chunk = 1 << shift
            pltpu.sync_copy(idx_hbm, idxv)

            def p1(j, n):
                v = idxv[pl.ds(16 * j, 16)]
                pos = iota16 + 16 * j
                m = (((v >> shift) & 31) == wid) & (v < vmax)
                mi = jnp.where(m, 1, 0)
                tgt = n + plsc.cumsum(mi) - mi
                plsc.store_scatter(lsti.at[:], [tgt], v, mask=m)
                plsc.store_scatter(lstp.at[:], [tgt], pos, mask=m)
                return n + jnp.sum(mi)

            n_my = lax.fori_loop(0, _B // 16, p1, 0)
            lsti[pl.ds(n_my, 16)] = jnp.full((16,), -1, jnp.int32)
            nv = (n_my + 15) >> 4
            nch_w = (nch - wid + 31) // 32
            fbase = []
            for g in range(ngroups):
                f = iota16 + 16 * g
                fbase.append((f >> 3, f & 7))

            def issue(ci, half, sem):
                base = pl.multiple_of(ci << shift, 128)
                for a in range(nadim):
                    pltpu.async_copy(
                        tbl.at[a, :, pl.ds(base, chunk)],
                        buf.at[half * nadim + a], sem)

            def wait_half(half, sem):
                for a in range(nadim):
                    pltpu.make_async_copy(
                        tbl.at[0, :, pl.ds(0, chunk)],
                        buf.at[half * nadim + a], sem).wait()

            def extract(c, half, k_in):
                def scan_body(j, k2):
                    v = lsti[pl.ds(16 * j, 16)]
                    vp = lstp[pl.ds(16 * j, 16)]
                    m = (v >> shift) == c
                    mi = jnp.where(m, 1, 0)
                    tgt = plsc.cumsum(mi) - mi
                    pk = (v & (chunk - 1)) | (vp << shift)
                    plsc.store_scatter(tmpi.at[:], [tgt], pk, mask=m)
                    km = jnp.sum(mi)

                    def ser(r, k3):
                        vt = tmpi[...]
                        sel = jnp.where(iota16 == r, 1, 0)
                        pk = jnp.sum(vt * sel)
                        off = pk & (chunk - 1)
                        pos = pk >> shift
                        slot = k3 & 15

                        @pl.when(k3 >= 16)
                        def _drain():
                            pltpu.make_async_copy(
                                out.at[pl.ds(0, 1)],
                                ring.at[pl.ds(slot, 1)], sem_out).wait()

                        offv = jnp.full((16,), off, jnp.int32)
                        for g in range(ngroups):
                            av, bv = fbase[g]
                            vals = plsc.load_gather(
                                buf.at[:, :, :],
                                [av + half * nadim, bv, offv])
                            ring[slot, pl.ds(16 * g, 16)] = vals
                        pltpu.async_copy(
                            ring.at[pl.ds(slot, 1)],
                            out.at[pl.ds(pos, 1)], sem_out)
                        return k3 + 1

                    return lax.fori_loop(0, km, ser, k2)

                return lax.fori_loop(0, nv, scan_body, k_in)

            issue(wid, 0, sem_sa)
            npairs = (nch_w + 1) // 2

            def pair_body(p2, k_out):
                c0 = wid + 64 * p2
                c1 = c0 + 32
                c2 = c0 + 64
                issue(jnp.minimum(c1, nch - 1), 1, sem_sb)
                wait_half(0, sem_sa)
                k_out = extract(c0, 0, k_out)
                issue(jnp.minimum(c2, nch - 1), 0, sem_sa)
                wait_half(1, sem_sb)
                return extract(c1, 1, k_out)

            k_out = lax.fori_loop(0, npairs, pair_body, k_out0)
            wait_half(0, sem_sa)
            return k_out

        k1 = phase(a_hbm, a3h, buf_a, out_a, _A_SHIFT, _A_NCH, _A_MAIN,
                   2, 4, 0)
        k2 = phase(c_hbm, t3h, buf_t, out_t, _T_SHIFT, _T_NCH, _T_MAIN,
                   4, 8, k1)
        for i in range(8):
            @pl.when(i < jnp.minimum(k2, 8))
            def _final_drain():
                pltpu.make_async_copy(
                    out_t.at[pl.ds(0, 1)], ring.at[pl.ds(0, 1)],
                    sem_out).wait()

    return gather_kernel(author, comment, a3, t3)


def _tc_s_body(su, s_emb, s_w, s_b, out):
    f32 = jnp.float32
    blk = su.shape[0]
    svocab = s_emb.shape[0]
    ia_s = lax.broadcasted_iota(jnp.int32, (blk, svocab), 1)
    oh_s = ((su[...][:, None] + ia_s * 0) == ia_s).astype(f32)
    se = jnp.dot(oh_s, s_emb[...], preferred_element_type=f32)
    out[...] = jnp.dot(se, s_w[...], preferred_element_type=f32) + s_b[...]


def _tc_s(su, s_emb, s_w, s_b):
    blk = 2048
    grid = _B // blk

    def full(x):
        return pl.BlockSpec(x.shape, lambda i: (0,) * x.ndim)

    return pl.pallas_call(
        _tc_s_body,
        grid=(grid,),
        in_specs=[pl.BlockSpec((blk,), lambda i: (i,)),
                  full(s_emb), full(s_w), full(s_b)],
        out_specs=pl.BlockSpec((blk, 50), lambda i: (i, 0)),
        out_shape=jax.ShapeDtypeStruct((_B, 50), jnp.float32),
    )(su, s_emb, s_w, s_b)


def _tc_body(au, cu, ga, gt, sru, a_tail, t_tail, a_w, a_b,
             t_w1, t_w2, t_b1, t_b2, l1a, l1c, l1_b, l2_w, l2_b, out):
    f32 = jnp.float32
    blk = au.shape[0]
    au_, cu_ = au[...], cu[...]
    ga_ = ga[...][:, :32]
    gt_ = gt[...][:, :64]
    ia32 = lax.broadcasted_iota(jnp.int32, (blk, 32), 1)
    au2 = au_[:, None] + ia32 * 0
    oh_a = ((au2 - _A_MAIN) == ia32).astype(f32)
    ae = jnp.where(au2 >= _A_MAIN,
                   jnp.dot(oh_a, a_tail[...], preferred_element_type=f32),
                   ga_)
    ia64 = lax.broadcasted_iota(jnp.int32, (blk, 64), 1)
    cu2 = cu_[:, None] + ia64 * 0
    oh_t = ((cu2 - _T_MAIN) == ia64).astype(f32)
    te = jnp.where(cu2 >= _T_MAIN,
                   jnp.dot(oh_t, t_tail[...], preferred_element_type=f32),
                   gt_)
    ar = jnp.dot(ae, a_w[...], preferred_element_type=f32) + a_b[...]
    sr = sru[...]
    cr1 = jnp.dot(te, t_w1[...], preferred_element_type=f32) + t_b1[...]
    cr2 = jnp.dot(te, t_w2[...], preferred_element_type=f32) + t_b2[...]
    m = (jnp.dot(ar * cr1, l1a[...], preferred_element_type=f32)
         + jnp.dot(sr * cr2, l1c[...], preferred_element_type=f32)
         + l1_b[...])
    o = jnp.dot(m, l2_w[...], preferred_element_type=f32) + l2_b[...]
    out[...] = o[:, 0]


def _tc_dense(au, cu, ga, gt, sru, *weights):
    blk = 2048
    grid = _B // blk

    def full(x):
        return pl.BlockSpec(x.shape, lambda i: (0,) * x.ndim)

    vec = pl.BlockSpec((blk,), lambda i: (i,))
    mat = pl.BlockSpec((blk, 128), lambda i: (i, 0))
    return pl.pallas_call(
        _tc_body,
        grid=(grid,),
        in_specs=[vec, vec, mat, mat,
                  pl.BlockSpec((blk, 50), lambda i: (i, 0)),
                  *[full(w) for w in weights]],
        out_specs=pl.BlockSpec((blk,), lambda i: (i,)),
        out_shape=jax.ShapeDtypeStruct((_B,), jnp.float32),
    )(au, cu, ga, gt, sru, *weights)


def kernel(author, subreddit, comment, A_emb, A_W, A_b, S_emb, S_W, S_b,
           T_emb, T_W, T_b, L1_W, L1_b, L2_W, L2_b):
    author = author.astype(jnp.int32)
    subreddit = subreddit.astype(jnp.int32)
    comment = comment.astype(jnp.int32)
    a3 = A_emb.T.reshape(4, 8, A_emb.shape[0])
    t3 = T_emb.T.reshape(8, 8, T_emb.shape[0])
    a_tail = A_emb[_A_MAIN:]
    t_tail = T_emb[_T_MAIN:]
    sru = _tc_s(subreddit, S_emb, S_W, S_b)
    ga, gt = _sc_gather(author, comment, a3, t3)
    t_w1, t_w2 = T_W[:, :50], T_W[:, 50:]
    t_b1, t_b2 = T_b[:50], T_b[50:]
    l1a, l1c = L1_W[:50, :], L1_W[50:, :]
    return _tc_dense(author, comment, ga, gt, sru,
                     a_tail, t_tail, A_W, A_b,
                     t_w1, t_w2, t_b1, t_b2, l1a, l1c, L1_b, L2_W, L2_b)


# speedup vs baseline: 1.1205x; 1.0004x over previous
import functools

import jax
import jax.numpy as jnp
from jax import lax
from jax.experimental import pallas as pl
from jax.experimental.pallas import tpu as pltpu
from jax.experimental.pallas import tpu_sc as plsc

_B = 16384
_A_MAIN = 99968
_T_MAIN = 999936
_A_SHIFT = 7
_T_SHIFT = 9
_A_NCH = _A_MAIN >> _A_SHIFT
_T_NCH = _T_MAIN >> _T_SHIFT


def _sc_gather(author, comment, a3, t3):
    info = plsc.get_sparse_core_info()
    nc, ns = info.num_cores, info.num_subcores
    mesh = plsc.VectorSubcoreMesh(core_axis_name="c", subcore_axis_name="s")

    @functools.partial(
        pl.kernel,
        mesh=mesh,
        compiler_params=pltpu.CompilerParams(needs_layout_passes=False),
        out_type=[
            jax.ShapeDtypeStruct((_B, 128), jnp.float32),
            jax.ShapeDtypeStruct((_B, 128), jnp.float32),
        ],
        scratch_types=[
            pltpu.VMEM((_B,), jnp.int32),
            pltpu.VMEM((_B + 16,), jnp.int32),
            pltpu.VMEM((_B + 16,), jnp.int32),
            pltpu.VMEM((8, 8, 128), jnp.float32),
            pltpu.VMEM((16, 8, 512), jnp.float32),
            pltpu.VMEM((16, 128), jnp.float32),
            pltpu.VMEM((16,), jnp.int32),
            pltpu.SemaphoreType.DMA,
            pltpu.SemaphoreType.DMA,
            pltpu.SemaphoreType.DMA,
        ],
    )
    def gather_kernel(a_hbm, c_hbm, a3h, t3h, out_a, out_t,
                      idxv, lsti, lstp, buf_a, buf_t, ring, tmpi,
                      sem_sa, sem_sb, sem_out):
        wid = lax.axis_index("s") * nc + lax.axis_index("c")
        iota16 = lax.broadcasted_iota(jnp.int32, (16,), 0)

        def phase(idx_hbm, tbl, buf, out, shift, nch, vmax, ngroups, nadim,
                  k_out0):
            chunk = 1 << shift
            pltpu.sync_copy(idx_hbm, idxv)

            def p1(j, n):
                v = idxv[pl.ds(16 * j, 16)]
                pos = iota16 + 16 * j
                m = (((v >> shift) & 31) == wid) & (v < vmax)
                mi = jnp.where(m, 1, 0)
                tgt = n + plsc.cumsum(mi) - mi
                plsc.store_scatter(lsti.at[:], [tgt], v, mask=m)
                plsc.store_scatter(lstp.at[:], [tgt], pos, mask=m)
                return n + jnp.sum(mi)

            n_my = lax.fori_loop(0, _B // 16, p1, 0)
            lsti[pl.ds(n_my, 16)] = jnp.full((16,), -1, jnp.int32)
            nv = (n_my + 15) >> 4
            nch_w = (nch - wid + 31) // 32
            fbase = []
            for g in range(ngroups):
                f = iota16 + 16 * g
                fbase.append((f >> 3, f & 7))

            def issue(ci, half, sem):
                base = pl.multiple_of(ci << shift, 128)
                for a in range(nadim):
                    pltpu.async_copy(
                        tbl.at[a, :, pl.ds(base, chunk)],
                        buf.at[half * nadim + a], sem)

            def wait_half(half, sem):
                for a in range(nadim):
                    pltpu.make_async_copy(
                        tbl.at[0, :, pl.ds(0, chunk)],
                        buf.at[half * nadim + a], sem).wait()

            def extract(c, half, k_in):
                def scan_body(j, k2):
                    v = lsti[pl.ds(16 * j, 16)]
                    vp = lstp[pl.ds(16 * j, 16)]
                    m = (v >> shift) == c
                    mi = jnp.where(m, 1, 0)
                    tgt = plsc.cumsum(mi) - mi
                    pk = (v & (chunk - 1)) | (vp << shift)
                    plsc.store_scatter(tmpi.at[:], [tgt], pk, mask=m)
                    km = jnp.sum(mi)

                    def ser(r, k3):
                        vt = tmpi[...]
                        sel = jnp.where(iota16 == r, 1, 0)
                        pk = jnp.sum(vt * sel)
                        off = pk & (chunk - 1)
                        pos = pk >> shift
                        slot = k3 & 15

                        @pl.when(k3 >= 16)
                        def _drain():
                            pltpu.make_async_copy(
                                out.at[pl.ds(0, 1)],
                                ring.at[pl.ds(slot, 1)], sem_out).wait()

                        offv = jnp.full((16,), off, jnp.int32)
                        for g in range(ngroups):
                            av, bv = fbase[g]
                            vals = plsc.load_gather(
                                buf.at[:, :, :],
                                [av + half * nadim, bv, offv])
                            ring[slot, pl.ds(16 * g, 16)] = vals
                        pltpu.async_copy(
                            ring.at[pl.ds(slot, 1)],
                            out.at[pl.ds(pos, 1)], sem_out)
                        return k3 + 1

                    return lax.fori_loop(0, km, ser, k2)

                return lax.fori_loop(0, nv, scan_body, k_in)

            issue(wid, 0, sem_sa)
            npairs = (nch_w + 1) // 2

            def pair_body(p2, k_out):
                c0 = wid + 64 * p2
                c1 = c0 + 32
                c2 = c0 + 64
                issue(jnp.minimum(c1, nch - 1), 1, sem_sb)
                wait_half(0, sem_sa)
                k_out = extract(c0, 0, k_out)
                issue(jnp.minimum(c2, nch - 1), 0, sem_sa)
                wait_half(1, sem_sb)
                return extract(c1, 1, k_out)

            k_out = lax.fori_loop(0, npairs, pair_body, k_out0)
            wait_half(0, sem_sa)
            return k_out

        k1 = phase(a_hbm, a3h, buf_a, out_a, _A_SHIFT, _A_NCH, _A_MAIN,
                   2, 4, 0)
        k2 = phase(c_hbm, t3h, buf_t, out_t, _T_SHIFT, _T_NCH, _T_MAIN,
                   4, 8, k1)
        for i in range(16):
            @pl.when(i < jnp.minimum(k2, 16))
            def _final_drain():
                pltpu.make_async_copy(
                    out_t.at[pl.ds(0, 1)], ring.at[pl.ds(0, 1)],
                    sem_out).wait()

    return gather_kernel(author, comment, a3, t3)


def _tc_s_body(su, s_emb, s_w, s_b, out):
    f32 = jnp.float32
    blk = su.shape[0]
    svocab = s_emb.shape[0]
    ia_s = lax.broadcasted_iota(jnp.int32, (blk, svocab), 1)
    oh_s = ((su[...][:, None] + ia_s * 0) == ia_s).astype(f32)
    se = jnp.dot(oh_s, s_emb[...], preferred_element_type=f32)
    out[...] = jnp.dot(se, s_w[...], preferred_element_type=f32) + s_b[...]


def _tc_s(su, s_emb, s_w, s_b):
    blk = 2048
    grid = _B // blk

    def full(x):
        return pl.BlockSpec(x.shape, lambda i: (0,) * x.ndim)

    return pl.pallas_call(
        _tc_s_body,
        grid=(grid,),
        in_specs=[pl.BlockSpec((blk,), lambda i: (i,)),
                  full(s_emb), full(s_w), full(s_b)],
        out_specs=pl.BlockSpec((blk, 50), lambda i: (i, 0)),
        out_shape=jax.ShapeDtypeStruct((_B, 50), jnp.float32),
    )(su, s_emb, s_w, s_b)


def _tc_body(au, cu, ga, gt, sru, a_tail, t_tail, a_w, a_b,
             t_w1, t_w2, t_b1, t_b2, l1a, l1c, l1_b, l2_w, l2_b, out):
    f32 = jnp.float32
    blk = au.shape[0]
    au_, cu_ = au[...], cu[...]
    ga_ = ga[...][:, :32]
    gt_ = gt[...][:, :64]
    ia32 = lax.broadcasted_iota(jnp.int32, (blk, 32), 1)
    au2 = au_[:, None] + ia32 * 0
    oh_a = ((au2 - _A_MAIN) == ia32).astype(f32)
    ae = jnp.where(au2 >= _A_MAIN,
                   jnp.dot(oh_a, a_tail[...], preferred_element_type=f32),
                   ga_)
    ia64 = lax.broadcasted_iota(jnp.int32, (blk, 64), 1)
    cu2 = cu_[:, None] + ia64 * 0
    oh_t = ((cu2 - _T_MAIN) == ia64).astype(f32)
    te = jnp.where(cu2 >= _T_MAIN,
                   jnp.dot(oh_t, t_tail[...], preferred_element_type=f32),
                   gt_)
    ar = jnp.dot(ae, a_w[...], preferred_element_type=f32) + a_b[...]
    sr = sru[...]
    cr1 = jnp.dot(te, t_w1[...], preferred_element_type=f32) + t_b1[...]
    cr2 = jnp.dot(te, t_w2[...], preferred_element_type=f32) + t_b2[...]
    m = (jnp.dot(ar * cr1, l1a[...], preferred_element_type=f32)
         + jnp.dot(sr * cr2, l1c[...], preferred_element_type=f32)
         + l1_b[...])
    o = jnp.dot(m, l2_w[...], preferred_element_type=f32) + l2_b[...]
    out[...] = o[:, 0]


def _tc_dense(au, cu, ga, gt, sru, *weights):
    blk = 2048
    grid = _B // blk

    def full(x):
        return pl.BlockSpec(x.shape, lambda i: (0,) * x.ndim)

    vec = pl.BlockSpec((blk,), lambda i: (i,))
    mat = pl.BlockSpec((blk, 128), lambda i: (i, 0))
    return pl.pallas_call(
        _tc_body,
        grid=(grid,),
        in_specs=[vec, vec, mat, mat,
                  pl.BlockSpec((blk, 50), lambda i: (i, 0)),
                  *[full(w) for w in weights]],
        out_specs=pl.BlockSpec((blk,), lambda i: (i,)),
        out_shape=jax.ShapeDtypeStruct((_B,), jnp.float32),
    )(au, cu, ga, gt, sru, *weights)


def kernel(author, subreddit, comment, A_emb, A_W, A_b, S_emb, S_W, S_b,
           T_emb, T_W, T_b, L1_W, L1_b, L2_W, L2_b):
    author = author.astype(jnp.int32)
    subreddit = subreddit.astype(jnp.int32)
    comment = comment.astype(jnp.int32)
    a3 = A_emb.T.reshape(4, 8, A_emb.shape[0])
    t3 = T_emb.T.reshape(8, 8, T_emb.shape[0])
    a_tail = A_emb[_A_MAIN:]
    t_tail = T_emb[_T_MAIN:]
    sru = _tc_s(subreddit, S_emb, S_W, S_b)
    ga, gt = _sc_gather(author, comment, a3, t3)
    t_w1, t_w2 = T_W[:, :50], T_W[:, 50:]
    t_b1, t_b2 = T_b[:50], T_b[50:]
    l1a, l1c = L1_W[:50, :], L1_W[50:, :]
    return _tc_dense(author, comment, ga, gt, sru,
                     a_tail, t_tail, A_W, A_b,
                     t_w1, t_w2, t_b1, t_b2, l1a, l1c, L1_b, L2_W, L2_b)
